# Initial kernel scaffold; baseline (speedup 1.0000x reference)
import jax
import jax.numpy as jnp
from jax.experimental import pallas as pl


def kernel(attribute_key, attribute_word, attribute_word_local_fw_pos, attribute_word_local_bw_pos, attribute_kv_pos, attribute_kw_pos, attribute_word_tag, field_key_table, field_word_table, local_pos_fw_table, local_pos_bw_table, kv_pos_table, kw_pos_table, field_tag_table):
    raise NotImplementedError("write your pallas kernel here")


# speedup vs baseline: 2.6926x; 2.6926x over previous
import jax
import jax.numpy as jnp
from jax import lax
from jax.experimental import pallas as pl
from jax.experimental.pallas import tpu as pltpu
from jax.experimental.pallas import tpu_sc as plsc

L_SEQ, B_SZ = 200, 1024
N_TOK = L_SEQ * B_SZ
NC, NS = 2, 16
NW = NC * NS
PER_W = N_TOK // NW
CHUNK = 128
NCH = PER_W // CHUNK
WIDTHS = (64, 64, 32, 32, 32, 32, 32)
COLS = (0, 64, 128, 160, 192, 224, 256)
OUT_D = 288
NT = 7


def _body(*refs):
    tables = refs[0:NT]
    idx_hbm = refs[NT:2 * NT]
    out = refs[2 * NT]
    idx_v = refs[2 * NT + 1:3 * NT + 1]
    rows = refs[3 * NT + 1:4 * NT + 1]
    sem_g = refs[4 * NT + 1]

    wid = lax.axis_index("s") * NC + lax.axis_index("c")

    for t in range(NT):
        pltpu.sync_copy(idx_hbm[t].at[wid], idx_v[t])

    def chunk_body(i, carry):
        cps = [
            pltpu.async_copy(tables[t].at[idx_v[t].at[i]], rows[t], sem_g)
            for t in range(NT)
        ]
        for cp in cps:
            cp.wait()
        for t in range(NT):
            pltpu.sync_copy(
                rows[t],
                out.at[wid, i, slice(None), pl.ds(COLS[t], WIDTHS[t])],
            )
        return carry

    lax.fori_loop(0, NCH, chunk_body, 0)


def kernel(attribute_key, attribute_word, attribute_word_local_fw_pos,
           attribute_word_local_bw_pos, attribute_kv_pos, attribute_kw_pos,
           attribute_word_tag, field_key_table, field_word_table,
           local_pos_fw_table, local_pos_bw_table, kv_pos_table,
           kw_pos_table, field_tag_table):
    tables = (field_word_table, field_key_table, local_pos_fw_table,
              local_pos_bw_table, kv_pos_table, kw_pos_table, field_tag_table)
    idx_arrays = (attribute_word, attribute_key, attribute_word_local_fw_pos,
                  attribute_word_local_bw_pos, attribute_kv_pos,
                  attribute_kw_pos, attribute_word_tag)
    idxs = [a.reshape(NW, NCH, CHUNK) for a in idx_arrays]

    mesh = plsc.VectorSubcoreMesh(core_axis_name="c", subcore_axis_name="s")
    scratch = (
        [pltpu.VMEM((NCH, CHUNK), jnp.int32) for _ in range(NT)]
        + [pltpu.VMEM((CHUNK, w), jnp.float32) for w in WIDTHS]
        + [pltpu.SemaphoreType.DMA]
    )
    out = pl.kernel(
        _body,
        out_type=jax.ShapeDtypeStruct((NW, NCH, CHUNK, OUT_D), jnp.float32),
        mesh=mesh,
        scratch_types=scratch,
        compiler_params=pltpu.CompilerParams(use_tc_tiling_on_sc=False),
    )(*tables, *idxs)
    return out.reshape(L_SEQ, B_SZ, OUT_D)
